# Initial kernel scaffold; baseline (speedup 1.0000x reference)
import jax
import jax.numpy as jnp
from jax.experimental import pallas as pl


def kernel(xin, yk):
    raise NotImplementedError("write your pallas kernel here")


# speedup vs baseline: 539.9593x; 539.9593x over previous
import functools

import jax
import jax.numpy as jnp
import numpy as np
from jax import lax
from jax.experimental import pallas as pl
from jax.experimental.pallas import tpu as pltpu
from jax.experimental.pallas import tpu_sc as plsc

N_TOK = 16384
N_FLT = 2048
N_KNOTS = 91
MAXX = 3.0
W = np.float32(2.0 * MAXX / (N_KNOTS - 1))
LO = np.float32(1e-5)
HI = np.float32(N_KNOTS - 1.00001)
MINX = np.float32(-MAXX)

NW = 32
CPW = 128
NSLAB = N_FLT // CPW
NROWH = NW // NSLAB
ROWS_PW = N_TOK // NROWH
LANES = 16
QPR = CPW // LANES
R = 128
NCHUNK = ROWS_PW // R


def _body(xin_hbm, yk_hbm, out_hbm, tab2d, tab_t, xbuf, ybuf):
    c = lax.axis_index("c")
    s = lax.axis_index("s")
    wid = s * 2 + c
    ch0 = (wid % NSLAB) * CPW
    rbase = (wid // NSLAB) * ROWS_PW

    iota = lax.iota(jnp.int32, LANES)
    cvec = [(iota + q * LANES) * N_KNOTS for q in range(QPR)]

    pltpu.sync_copy(yk_hbm.at[:, pl.ds(ch0, CPW)], tab2d)

    def t_body(r, carry):
        for q in range(QPR):
            v = tab2d[r, pl.ds(q * LANES, LANES)]
            plsc.store_scatter(tab_t, [cvec[q] + r], v)
        return carry

    lax.fori_loop(0, N_KNOTS, t_body, 0)

    def chunk_body(ci, carry):
        r0 = rbase + ci * R
        pltpu.sync_copy(xin_hbm.at[pl.ds(r0, R), pl.ds(ch0, CPW)], xbuf)

        def row_body(r, carry2):
            for q in range(QPR):
                x = xbuf[r, pl.ds(q * LANES, LANES)]
                xs = (x - MINX) / W
                xs = jnp.minimum(jnp.maximum(xs, LO), HI)
                idxf = xs.astype(jnp.int32)
                k = xs - idxf.astype(jnp.float32)
                flat = cvec[q] + idxf
                yf = plsc.load_gather(tab_t, [flat])
                yc = plsc.load_gather(tab_t, [flat + 1])
                ybuf[r, pl.ds(q * LANES, LANES)] = yf + k * (yc - yf)
            return carry2

        lax.fori_loop(0, R, row_body, 0, unroll=2)
        pltpu.sync_copy(ybuf, out_hbm.at[pl.ds(r0, R), pl.ds(ch0, CPW)])
        return carry

    lax.fori_loop(0, NCHUNK, chunk_body, 0)


@jax.jit
def kernel(xin, yk):
    run = pl.kernel(
        _body,
        out_type=jax.ShapeDtypeStruct((N_TOK, N_FLT), jnp.float32),
        mesh=plsc.VectorSubcoreMesh(core_axis_name="c", subcore_axis_name="s"),
        compiler_params=pltpu.CompilerParams(needs_layout_passes=False),
        scratch_types=[
            pltpu.VMEM((N_KNOTS, CPW), jnp.float32),
            pltpu.VMEM((CPW * N_KNOTS,), jnp.float32),
            pltpu.VMEM((R, CPW), jnp.float32),
            pltpu.VMEM((R, CPW), jnp.float32),
        ],
    )
    return run(xin, yk)


# speedup vs baseline: 590.9755x; 1.0945x over previous
import functools

import jax
import jax.numpy as jnp
import numpy as np
from jax import lax
from jax.experimental import pallas as pl
from jax.experimental.pallas import tpu as pltpu
from jax.experimental.pallas import tpu_sc as plsc

N_TOK = 16384
N_FLT = 2048
N_KNOTS = 91
MAXX = 3.0
W = np.float32(2.0 * MAXX / (N_KNOTS - 1))
LO = np.float32(1e-5)
HI = np.float32(N_KNOTS - 1.00001)
MINX = np.float32(-MAXX)

NW = 32
CPW = 128
NSLAB = N_FLT // CPW
NROWH = NW // NSLAB
ROWS_PW = N_TOK // NROWH
LANES = 16
QPR = CPW // LANES
R = 128
NCHUNK = ROWS_PW // R


def _body(xin_hbm, yk_hbm, out_hbm, tab2d, tab_t, xb0, xb1, yb0, yb1,
          isem0, isem1, osem0, osem1):
    c = lax.axis_index("c")
    s = lax.axis_index("s")
    wid = s * 2 + c
    ch0 = (wid % NSLAB) * CPW
    rbase = (wid // NSLAB) * ROWS_PW

    iota = lax.iota(jnp.int32, LANES)
    cvec = [(iota + q * LANES) * N_KNOTS for q in range(QPR)]

    def in_slice(ci):
        return xin_hbm.at[pl.ds(rbase + ci * R, R), pl.ds(ch0, CPW)]

    def out_slice(ci):
        return out_hbm.at[pl.ds(rbase + ci * R, R), pl.ds(ch0, CPW)]

    def start_in(ci, buf, sem):
        pltpu.async_copy(in_slice(ci), buf, sem)

    def wait_in(ci, buf, sem):
        pltpu.make_async_copy(in_slice(ci), buf, sem).wait()

    def start_out(ci, buf, sem):
        pltpu.async_copy(buf, out_slice(ci), sem)

    def wait_out(ci, buf, sem):
        pltpu.make_async_copy(buf, out_slice(ci), sem).wait()

    def compute(xbuf, ybuf):
        def row_body(r, carry2):
            for q in range(QPR):
                x = xbuf[r, pl.ds(q * LANES, LANES)]
                xs = (x - MINX) / W
                xs = jnp.minimum(jnp.maximum(xs, LO), HI)
                idxf = xs.astype(jnp.int32)
                k = xs - idxf.astype(jnp.float32)
                flat = cvec[q] + idxf
                yf = plsc.load_gather(tab_t, [flat])
                yc = plsc.load_gather(tab_t, [flat + 1])
                ybuf[r, pl.ds(q * LANES, LANES)] = yf + k * (yc - yf)
            return carry2

        lax.fori_loop(0, R, row_body, 0, unroll=2)

    start_in(0, xb0, isem0)
    start_in(1, xb1, isem1)

    pltpu.sync_copy(yk_hbm.at[:, pl.ds(ch0, CPW)], tab2d)

    def t_body(r, carry):
        for q in range(QPR):
            v = tab2d[r, pl.ds(q * LANES, LANES)]
            plsc.store_scatter(tab_t, [cvec[q] + r], v)
        return carry

    lax.fori_loop(0, N_KNOTS, t_body, 0)

    wait_in(0, xb0, isem0)
    compute(xb0, yb0)
    start_out(0, yb0, osem0)
    start_in(2, xb0, isem0)
    wait_in(1, xb1, isem1)
    compute(xb1, yb1)
    start_out(1, yb1, osem1)
    start_in(3, xb1, isem1)

    def chunk_body(i, carry):
        c0 = 2 * i
        wait_in(c0, xb0, isem0)
        wait_out(c0 - 2, yb0, osem0)
        compute(xb0, yb0)
        start_out(c0, yb0, osem0)
        start_in(c0 + 2, xb0, isem0)
        wait_in(c0 + 1, xb1, isem1)
        wait_out(c0 - 1, yb1, osem1)
        compute(xb1, yb1)
        start_out(c0 + 1, yb1, osem1)
        start_in(c0 + 3, xb1, isem1)
        return carry

    lax.fori_loop(1, NCHUNK // 2 - 1, chunk_body, 0)

    c0 = NCHUNK - 2
    wait_in(c0, xb0, isem0)
    wait_out(c0 - 2, yb0, osem0)
    compute(xb0, yb0)
    start_out(c0, yb0, osem0)
    wait_in(c0 + 1, xb1, isem1)
    wait_out(c0 - 1, yb1, osem1)
    compute(xb1, yb1)
    start_out(c0 + 1, yb1, osem1)
    wait_out(c0, yb0, osem0)
    wait_out(c0 + 1, yb1, osem1)


@jax.jit
def kernel(xin, yk):
    run = pl.kernel(
        _body,
        out_type=jax.ShapeDtypeStruct((N_TOK, N_FLT), jnp.float32),
        mesh=plsc.VectorSubcoreMesh(core_axis_name="c", subcore_axis_name="s"),
        compiler_params=pltpu.CompilerParams(needs_layout_passes=False),
        scratch_types=[
            pltpu.VMEM((N_KNOTS, CPW), jnp.float32),
            pltpu.VMEM((CPW * N_KNOTS,), jnp.float32),
            pltpu.VMEM((R, CPW), jnp.float32),
            pltpu.VMEM((R, CPW), jnp.float32),
            pltpu.VMEM((R, CPW), jnp.float32),
            pltpu.VMEM((R, CPW), jnp.float32),
            pltpu.SemaphoreType.DMA,
            pltpu.SemaphoreType.DMA,
            pltpu.SemaphoreType.DMA,
            pltpu.SemaphoreType.DMA,
        ],
    )
    return run(xin, yk)


# speedup vs baseline: 3294.6383x; 5.5749x over previous
import functools

import jax
import jax.numpy as jnp
import numpy as np
from jax import lax
from jax.experimental import pallas as pl
from jax.experimental.pallas import tpu as pltpu
from jax.experimental.pallas import tpu_sc as plsc

N_TOK = 16384
N_FLT = 2048
N_KNOTS = 91
MAXX = 3.0
W = np.float32(2.0 * MAXX / (N_KNOTS - 1))
LO = np.float32(1e-5)
HI = np.float32(N_KNOTS - 1.00001)
MINX = np.float32(-MAXX)

NW = 32
CPW = 128
NSLAB = N_FLT // CPW
NROWH = NW // NSLAB
ROWS_PW = N_TOK // NROWH
LANES = 16
QPR = CPW // LANES
R = 128
NCHUNK = ROWS_PW // R


def _body(xin_hbm, yk_hbm, out_hbm, tab2d, tab_t, xb0, xb1, yb0, yb1,
          isem0, isem1, osem0, osem1):
    c = lax.axis_index("c")
    s = lax.axis_index("s")
    wid = s * 2 + c
    ch0 = (wid % NSLAB) * CPW
    rbase = (wid // NSLAB) * ROWS_PW

    iota = lax.iota(jnp.int32, LANES)
    cvec = [(iota + q * LANES) * N_KNOTS for q in range(QPR)]

    def in_slice(ci):
        return xin_hbm.at[pl.ds(rbase + ci * R, R), pl.ds(ch0, CPW)]

    def out_slice(ci):
        return out_hbm.at[pl.ds(rbase + ci * R, R), pl.ds(ch0, CPW)]

    def start_in(ci, buf, sem):
        pltpu.async_copy(in_slice(ci), buf, sem)

    def wait_in(ci, buf, sem):
        pltpu.make_async_copy(in_slice(ci), buf, sem).wait()

    def start_out(ci, buf, sem):
        pltpu.async_copy(buf, out_slice(ci), sem)

    def wait_out(ci, buf, sem):
        pltpu.make_async_copy(buf, out_slice(ci), sem).wait()

    def compute(xbuf, ybuf):
        @plsc.parallel_loop(0, R, unroll=2)
        def row_body(r):
            for q in range(QPR):
                x = xbuf[r, pl.ds(q * LANES, LANES)]
                xs = (x - MINX) / W
                xs = jnp.minimum(jnp.maximum(xs, LO), HI)
                idxf = xs.astype(jnp.int32)
                k = xs - idxf.astype(jnp.float32)
                flat = cvec[q] + idxf
                yf = plsc.load_gather(tab_t, [flat])
                yc = plsc.load_gather(tab_t, [flat + 1])
                ybuf[r, pl.ds(q * LANES, LANES)] = yf + k * (yc - yf)

    start_in(0, xb0, isem0)
    start_in(1, xb1, isem1)

    pltpu.sync_copy(yk_hbm.at[:, pl.ds(ch0, CPW)], tab2d)

    def t_body(r, carry):
        for q in range(QPR):
            v = tab2d[r, pl.ds(q * LANES, LANES)]
            plsc.store_scatter(tab_t, [cvec[q] + r], v)
        return carry

    lax.fori_loop(0, N_KNOTS, t_body, 0)

    wait_in(0, xb0, isem0)
    compute(xb0, yb0)
    start_out(0, yb0, osem0)
    start_in(2, xb0, isem0)
    wait_in(1, xb1, isem1)
    compute(xb1, yb1)
    start_out(1, yb1, osem1)
    start_in(3, xb1, isem1)

    def chunk_body(i, carry):
        c0 = 2 * i
        wait_in(c0, xb0, isem0)
        wait_out(c0 - 2, yb0, osem0)
        compute(xb0, yb0)
        start_out(c0, yb0, osem0)
        start_in(c0 + 2, xb0, isem0)
        wait_in(c0 + 1, xb1, isem1)
        wait_out(c0 - 1, yb1, osem1)
        compute(xb1, yb1)
        start_out(c0 + 1, yb1, osem1)
        start_in(c0 + 3, xb1, isem1)
        return carry

    lax.fori_loop(1, NCHUNK // 2 - 1, chunk_body, 0)

    c0 = NCHUNK - 2
    wait_in(c0, xb0, isem0)
    wait_out(c0 - 2, yb0, osem0)
    compute(xb0, yb0)
    start_out(c0, yb0, osem0)
    wait_in(c0 + 1, xb1, isem1)
    wait_out(c0 - 1, yb1, osem1)
    compute(xb1, yb1)
    start_out(c0 + 1, yb1, osem1)
    wait_out(c0, yb0, osem0)
    wait_out(c0 + 1, yb1, osem1)


@jax.jit
def kernel(xin, yk):
    run = pl.kernel(
        _body,
        out_type=jax.ShapeDtypeStruct((N_TOK, N_FLT), jnp.float32),
        mesh=plsc.VectorSubcoreMesh(core_axis_name="c", subcore_axis_name="s"),
        compiler_params=pltpu.CompilerParams(needs_layout_passes=False),
        scratch_types=[
            pltpu.VMEM((N_KNOTS, CPW), jnp.float32),
            pltpu.VMEM((CPW * N_KNOTS,), jnp.float32),
            pltpu.VMEM((R, CPW), jnp.float32),
            pltpu.VMEM((R, CPW), jnp.float32),
            pltpu.VMEM((R, CPW), jnp.float32),
            pltpu.VMEM((R, CPW), jnp.float32),
            pltpu.SemaphoreType.DMA,
            pltpu.SemaphoreType.DMA,
            pltpu.SemaphoreType.DMA,
            pltpu.SemaphoreType.DMA,
        ],
    )
    return run(xin, yk)


# speedup vs baseline: 3389.7935x; 1.0289x over previous
import functools

import jax
import jax.numpy as jnp
import numpy as np
from jax import lax
from jax.experimental import pallas as pl
from jax.experimental.pallas import tpu as pltpu
from jax.experimental.pallas import tpu_sc as plsc

N_TOK = 16384
N_FLT = 2048
N_KNOTS = 91
MAXX = 3.0
W = np.float32(2.0 * MAXX / (N_KNOTS - 1))
LO = np.float32(1e-5)
HI = np.float32(N_KNOTS - 1.00001)
MINX = np.float32(-MAXX)

NW = 32
CPW = 128
NSLAB = N_FLT // CPW
NROWH = NW // NSLAB
ROWS_PW = N_TOK // NROWH
LANES = 16
QPR = CPW // LANES
R = 128
NCHUNK = ROWS_PW // R


def _body(xin_hbm, yk_hbm, out_hbm, tab2d, tab_t, xb0, xb1, yb0, yb1,
          isem0, isem1, osem0, osem1):
    c = lax.axis_index("c")
    s = lax.axis_index("s")
    wid = s * 2 + c
    ch0 = (wid % NSLAB) * CPW
    rbase = (wid // NSLAB) * ROWS_PW

    iota = lax.iota(jnp.int32, LANES)
    cvec = [(iota + q * LANES) * N_KNOTS for q in range(QPR)]

    def in_slice(ci):
        return xin_hbm.at[pl.ds(rbase + ci * R, R), pl.ds(ch0, CPW)]

    def out_slice(ci):
        return out_hbm.at[pl.ds(rbase + ci * R, R), pl.ds(ch0, CPW)]

    def start_in(ci, buf, sem):
        pltpu.async_copy(in_slice(ci), buf, sem)

    def wait_in(ci, buf, sem):
        pltpu.make_async_copy(in_slice(ci), buf, sem).wait()

    def start_out(ci, buf, sem):
        pltpu.async_copy(buf, out_slice(ci), sem)

    def wait_out(ci, buf, sem):
        pltpu.make_async_copy(buf, out_slice(ci), sem).wait()

    def compute(xbuf, ybuf):
        @plsc.parallel_loop(0, R, unroll=4)
        def row_body(r):
            for q in range(QPR):
                x = xbuf[r, pl.ds(q * LANES, LANES)]
                xs = (x - MINX) / W
                xs = jnp.minimum(jnp.maximum(xs, LO), HI)
                idxf = xs.astype(jnp.int32)
                k = xs - idxf.astype(jnp.float32)
                flat = cvec[q] + idxf
                yf = plsc.load_gather(tab_t, [flat])
                yc = plsc.load_gather(tab_t, [flat + 1])
                ybuf[r, pl.ds(q * LANES, LANES)] = yf + k * (yc - yf)

    start_in(0, xb0, isem0)
    start_in(1, xb1, isem1)

    pltpu.sync_copy(yk_hbm.at[:, pl.ds(ch0, CPW)], tab2d)

    def t_body(r, carry):
        for q in range(QPR):
            v = tab2d[r, pl.ds(q * LANES, LANES)]
            plsc.store_scatter(tab_t, [cvec[q] + r], v)
        return carry

    lax.fori_loop(0, N_KNOTS, t_body, 0)

    wait_in(0, xb0, isem0)
    compute(xb0, yb0)
    start_out(0, yb0, osem0)
    start_in(2, xb0, isem0)
    wait_in(1, xb1, isem1)
    compute(xb1, yb1)
    start_out(1, yb1, osem1)
    start_in(3, xb1, isem1)

    def chunk_body(i, carry):
        c0 = 2 * i
        wait_in(c0, xb0, isem0)
        wait_out(c0 - 2, yb0, osem0)
        compute(xb0, yb0)
        start_out(c0, yb0, osem0)
        start_in(c0 + 2, xb0, isem0)
        wait_in(c0 + 1, xb1, isem1)
        wait_out(c0 - 1, yb1, osem1)
        compute(xb1, yb1)
        start_out(c0 + 1, yb1, osem1)
        start_in(c0 + 3, xb1, isem1)
        return carry

    lax.fori_loop(1, NCHUNK // 2 - 1, chunk_body, 0)

    c0 = NCHUNK - 2
    wait_in(c0, xb0, isem0)
    wait_out(c0 - 2, yb0, osem0)
    compute(xb0, yb0)
    start_out(c0, yb0, osem0)
    wait_in(c0 + 1, xb1, isem1)
    wait_out(c0 - 1, yb1, osem1)
    compute(xb1, yb1)
    start_out(c0 + 1, yb1, osem1)
    wait_out(c0, yb0, osem0)
    wait_out(c0 + 1, yb1, osem1)


@jax.jit
def kernel(xin, yk):
    run = pl.kernel(
        _body,
        out_type=jax.ShapeDtypeStruct((N_TOK, N_FLT), jnp.float32),
        mesh=plsc.VectorSubcoreMesh(core_axis_name="c", subcore_axis_name="s"),
        compiler_params=pltpu.CompilerParams(needs_layout_passes=False),
        scratch_types=[
            pltpu.VMEM((N_KNOTS, CPW), jnp.float32),
            pltpu.VMEM((CPW * N_KNOTS,), jnp.float32),
            pltpu.VMEM((R, CPW), jnp.float32),
            pltpu.VMEM((R, CPW), jnp.float32),
            pltpu.VMEM((R, CPW), jnp.float32),
            pltpu.VMEM((R, CPW), jnp.float32),
            pltpu.SemaphoreType.DMA,
            pltpu.SemaphoreType.DMA,
            pltpu.SemaphoreType.DMA,
            pltpu.SemaphoreType.DMA,
        ],
    )
    return run(xin, yk)


# speedup vs baseline: 3465.5642x; 1.0224x over previous
import functools

import jax
import jax.numpy as jnp
import numpy as np
from jax import lax
from jax.experimental import pallas as pl
from jax.experimental.pallas import tpu as pltpu
from jax.experimental.pallas import tpu_sc as plsc

N_TOK = 16384
N_FLT = 2048
N_KNOTS = 91
MAXX = 3.0
W = np.float32(2.0 * MAXX / (N_KNOTS - 1))
WINV = np.float32(1.0) / W
LO = np.float32(1e-5)
HI = np.float32(N_KNOTS - 1.00001)
MINX = np.float32(-MAXX)

NW = 32
CPW = 128
NSLAB = N_FLT // CPW
NROWH = NW // NSLAB
ROWS_PW = N_TOK // NROWH
LANES = 16
QPR = CPW // LANES
R = 128
NCHUNK = ROWS_PW // R


def _body(xin_hbm, yk_hbm, out_hbm, tab2d, tab_a, tab_d, xb0, xb1, yb0, yb1,
          isem0, isem1, osem0, osem1):
    c = lax.axis_index("c")
    s = lax.axis_index("s")
    wid = s * 2 + c
    ch0 = (wid % NSLAB) * CPW
    rbase = (wid // NSLAB) * ROWS_PW

    iota = lax.iota(jnp.int32, LANES)
    cvec = [(iota + q * LANES) * N_KNOTS for q in range(QPR)]

    def in_slice(ci):
        return xin_hbm.at[pl.ds(rbase + ci * R, R), pl.ds(ch0, CPW)]

    def out_slice(ci):
        return out_hbm.at[pl.ds(rbase + ci * R, R), pl.ds(ch0, CPW)]

    def start_in(ci, buf, sem):
        pltpu.async_copy(in_slice(ci), buf, sem)

    def wait_in(ci, buf, sem):
        pltpu.make_async_copy(in_slice(ci), buf, sem).wait()

    def start_out(ci, buf, sem):
        pltpu.async_copy(buf, out_slice(ci), sem)

    def wait_out(ci, buf, sem):
        pltpu.make_async_copy(buf, out_slice(ci), sem).wait()

    def compute(xbuf, ybuf):
        @plsc.parallel_loop(0, R, unroll=4)
        def row_body(r):
            for q in range(QPR):
                x = xbuf[r, pl.ds(q * LANES, LANES)]
                xs = (x - MINX) * WINV
                xs = jnp.minimum(jnp.maximum(xs, LO), HI)
                idxf = xs.astype(jnp.int32)
                k = xs - idxf.astype(jnp.float32)
                flat = cvec[q] + idxf
                a = plsc.load_gather(tab_a, [flat])
                d = plsc.load_gather(tab_d, [flat])
                ybuf[r, pl.ds(q * LANES, LANES)] = a + k * d

    start_in(0, xb0, isem0)
    start_in(1, xb1, isem1)

    pltpu.sync_copy(yk_hbm.at[:, pl.ds(ch0, CPW)], tab2d)

    def t_body(r, carry):
        for q in range(QPR):
            v = tab2d[r, pl.ds(q * LANES, LANES)]
            vn = tab2d[r + 1, pl.ds(q * LANES, LANES)]
            plsc.store_scatter(tab_a, [cvec[q] + r], v)
            plsc.store_scatter(tab_d, [cvec[q] + r], vn - v)
        return carry

    lax.fori_loop(0, N_KNOTS - 1, t_body, 0)
    for q in range(QPR):
        v = tab2d[N_KNOTS - 1, pl.ds(q * LANES, LANES)]
        plsc.store_scatter(tab_a, [cvec[q] + (N_KNOTS - 1)], v)

    wait_in(0, xb0, isem0)
    compute(xb0, yb0)
    start_out(0, yb0, osem0)
    start_in(2, xb0, isem0)
    wait_in(1, xb1, isem1)
    compute(xb1, yb1)
    start_out(1, yb1, osem1)
    start_in(3, xb1, isem1)

    def chunk_body(i, carry):
        c0 = 2 * i
        wait_in(c0, xb0, isem0)
        wait_out(c0 - 2, yb0, osem0)
        compute(xb0, yb0)
        start_out(c0, yb0, osem0)
        start_in(c0 + 2, xb0, isem0)
        wait_in(c0 + 1, xb1, isem1)
        wait_out(c0 - 1, yb1, osem1)
        compute(xb1, yb1)
        start_out(c0 + 1, yb1, osem1)
        start_in(c0 + 3, xb1, isem1)
        return carry

    lax.fori_loop(1, NCHUNK // 2 - 1, chunk_body, 0)

    c0 = NCHUNK - 2
    wait_in(c0, xb0, isem0)
    wait_out(c0 - 2, yb0, osem0)
    compute(xb0, yb0)
    start_out(c0, yb0, osem0)
    wait_in(c0 + 1, xb1, isem1)
    wait_out(c0 - 1, yb1, osem1)
    compute(xb1, yb1)
    start_out(c0 + 1, yb1, osem1)
    wait_out(c0, yb0, osem0)
    wait_out(c0 + 1, yb1, osem1)


@jax.jit
def kernel(xin, yk):
    run = pl.kernel(
        _body,
        out_type=jax.ShapeDtypeStruct((N_TOK, N_FLT), jnp.float32),
        mesh=plsc.VectorSubcoreMesh(core_axis_name="c", subcore_axis_name="s"),
        compiler_params=pltpu.CompilerParams(needs_layout_passes=False),
        scratch_types=[
            pltpu.VMEM((N_KNOTS, CPW), jnp.float32),
            pltpu.VMEM((CPW * N_KNOTS,), jnp.float32),
            pltpu.VMEM((CPW * N_KNOTS,), jnp.float32),
            pltpu.VMEM((R, CPW), jnp.float32),
            pltpu.VMEM((R, CPW), jnp.float32),
            pltpu.VMEM((R, CPW), jnp.float32),
            pltpu.VMEM((R, CPW), jnp.float32),
            pltpu.SemaphoreType.DMA,
            pltpu.SemaphoreType.DMA,
            pltpu.SemaphoreType.DMA,
            pltpu.SemaphoreType.DMA,
        ],
    )
    return run(xin, yk)
